# baseline (speedup 1.0000x reference)
import jax
import jax.numpy as jnp
from jax import lax
from jax.experimental import pallas as pl

_N = 700
_NP = 704
_HEADS = 8
_HID = 8
_XD = 128

_TT = (((1,), (1,)), ((), ()))


def _leaky(v):
    return jnp.maximum(v, 0.2 * v)


def _attend(g, a_dst_row, a_src_row, pad_mask_row):
    ad_col = lax.dot_general(g, a_dst_row, _TT,
                             preferred_element_type=jnp.float32)
    as_row = lax.dot_general(a_src_row, g, _TT,
                             preferred_element_type=jnp.float32)
    as_row = jnp.where(pad_mask_row, -1e31, as_row)
    as_max = jnp.max(as_row, axis=1, keepdims=True)
    m_col = _leaky(ad_col + as_max)
    p = jnp.exp(_leaky(ad_col + as_row) - m_col)
    denom = jnp.sum(p, axis=1, keepdims=True)
    num = jnp.dot(p, g, preferred_element_type=jnp.float32)
    return num / (denom + 1e-16)


def _gat_body(x_ref, w1_ref, as1_ref, ad1_ref, b1_ref,
              w2_ref, as2_ref, ad2_ref, b2_ref, o_ref):
    pad_row = jax.lax.broadcasted_iota(jnp.int32, (1, _NP), 1) >= _N

    h = jnp.dot(x_ref[...], w1_ref[...], preferred_element_type=jnp.float32)
    outs = []
    for i in range(_HEADS):
        hi = h[:, i * _HID:(i + 1) * _HID]
        outs.append(_attend(hi, ad1_ref[i:i + 1, :], as1_ref[i:i + 1, :],
                            pad_row))
    h1 = jnp.concatenate(outs, axis=1) + b1_ref[...]
    h1 = jnp.maximum(h1, 0.0)

    g = jnp.dot(h1, w2_ref[...], preferred_element_type=jnp.float32)
    o_ref[...] = _attend(g, ad2_ref[...], as2_ref[...], pad_row) + b2_ref[...]


def kernel(x, W1, a_src1, a_dst1, b1, W2, a_src2, a_dst2, b2):
    x_p = jnp.zeros((_NP, _XD), jnp.float32).at[:_N, :].set(x)
    out = pl.pallas_call(
        _gat_body,
        out_shape=jax.ShapeDtypeStruct((_NP, _XD), jnp.float32),
    )(x_p, W1, a_src1, a_dst1, b1.reshape(1, -1),
      W2, a_src2, a_dst2, b2.reshape(1, -1))
    return out[:_N, :]


# speedup vs baseline: 1.1044x; 1.1044x over previous
import jax
import jax.numpy as jnp
from jax import lax
from jax.experimental import pallas as pl

_N = 700
_NP = 704
_HEADS = 8
_HID = 8
_XD = 128

_TT = (((1,), (1,)), ((), ()))


def _leaky(v):
    return jnp.maximum(v, 0.2 * v)


def _attend(g, a_dst_row, a_src_row, pad_mask_row):
    ad_col = jnp.sum(g * a_dst_row, axis=1, keepdims=True)
    as_row = jnp.sum(g * a_src_row, axis=1).reshape(1, _NP)
    as_row = jnp.where(pad_mask_row, -1e31, as_row)
    as_max = jnp.max(as_row, axis=1, keepdims=True)
    m_col = _leaky(ad_col + as_max)
    p = jnp.exp(_leaky(ad_col + as_row) - m_col)
    denom = jnp.sum(p, axis=1, keepdims=True)
    num = jnp.dot(p, g, preferred_element_type=jnp.float32)
    return num / (denom + 1e-16)


def _gat_body(x_ref, w1_ref, as1_ref, ad1_ref, b1_ref,
              w2_ref, as2_ref, ad2_ref, b2_ref, o_ref):
    pad_row = jax.lax.broadcasted_iota(jnp.int32, (1, _NP), 1) >= _N

    h = jnp.dot(x_ref[...], w1_ref[...], preferred_element_type=jnp.float32)
    outs = []
    for i in range(_HEADS):
        hi = h[:, i * _HID:(i + 1) * _HID]
        outs.append(_attend(hi, ad1_ref[i:i + 1, :], as1_ref[i:i + 1, :],
                            pad_row))
    h1 = jnp.concatenate(outs, axis=1) + b1_ref[...]
    h1 = jnp.maximum(h1, 0.0)

    g = jnp.dot(h1, w2_ref[...], preferred_element_type=jnp.float32)
    o_ref[...] = _attend(g, ad2_ref[...], as2_ref[...], pad_row) + b2_ref[...]


def kernel(x, W1, a_src1, a_dst1, b1, W2, a_src2, a_dst2, b2):
    x_p = jnp.zeros((_NP, _XD), jnp.float32).at[:_N, :].set(x)
    out = pl.pallas_call(
        _gat_body,
        out_shape=jax.ShapeDtypeStruct((_NP, _XD), jnp.float32),
    )(x_p, W1, a_src1, a_dst1, b1.reshape(1, -1),
      W2, a_src2, a_dst2, b2.reshape(1, -1))
    return out[:_N, :]


# speedup vs baseline: 1.2160x; 1.1010x over previous
import jax
import jax.numpy as jnp
from jax.experimental import pallas as pl

_N = 700
_HEADS = 8
_HID = 8
_XD = 128


def _leaky(v):
    return jnp.maximum(v, 0.2 * v)


def _attend(g, a_dst_row, a_src_row):
    ad_col = jnp.sum(g * a_dst_row, axis=1, keepdims=True)
    as_row = jnp.sum(g * a_src_row, axis=1).reshape(1, _N)
    as_max = jnp.max(as_row, axis=1, keepdims=True)
    m_col = _leaky(ad_col + as_max)
    p = jnp.exp(_leaky(ad_col + as_row) - m_col)
    denom = jnp.sum(p, axis=1, keepdims=True)
    num = jnp.dot(p, g, preferred_element_type=jnp.float32)
    return num / (denom + 1e-16)


def _gat_body(x_ref, w1_ref, as1_ref, ad1_ref, b1_ref,
              w2_ref, as2_ref, ad2_ref, b2_ref, o_ref):
    h = jnp.dot(x_ref[...], w1_ref[...], preferred_element_type=jnp.float32)
    outs = []
    for i in range(_HEADS):
        hi = h[:, i * _HID:(i + 1) * _HID]
        outs.append(_attend(hi, ad1_ref[i:i + 1, :], as1_ref[i:i + 1, :]))
    h1 = jnp.concatenate(outs, axis=1) + b1_ref[...]
    h1 = jnp.maximum(h1, 0.0)

    g = jnp.dot(h1, w2_ref[...], preferred_element_type=jnp.float32)
    o_ref[...] = _attend(g, ad2_ref[...], as2_ref[...]) + b2_ref[...]


def kernel(x, W1, a_src1, a_dst1, b1, W2, a_src2, a_dst2, b2):
    return pl.pallas_call(
        _gat_body,
        out_shape=jax.ShapeDtypeStruct((_N, _XD), jnp.float32),
    )(x, W1, a_src1, a_dst1, b1.reshape(1, -1),
      W2, a_src2, a_dst2, b2.reshape(1, -1))


# speedup vs baseline: 1.6104x; 1.3244x over previous
import jax
import jax.numpy as jnp
from jax.experimental import pallas as pl

_N = 700
_HEADS = 8
_HID = 8
_XD = 128
_LOG2E = 1.4426950408889634


def _leaky(v):
    return jnp.maximum(v, 0.2 * v)


def _attend(g_aug, ad_col, as_row, as_max):
    c = _leaky(ad_col + as_max)
    col1 = ad_col - c
    col2 = 0.2 * ad_col - c
    row2 = 0.2 * as_row
    p = jnp.exp2(jnp.maximum(col1 + as_row, col2 + row2))
    aug = jnp.dot(p, g_aug, preferred_element_type=jnp.float32)
    w = g_aug.shape[1] - 1
    return aug[:, :w], aug[:, w:w + 1]


def _gat_body(x_ref, w1_ref, bdas_ref, bdad_ref, b1_ref,
              w2_ref, as2_ref, ad2_ref, b2_ref, o_ref):
    ones_col = jnp.ones((_N, 1), jnp.float32)

    h = jnp.dot(x_ref[...], w1_ref[...], preferred_element_type=jnp.float32)
    ad_all = jnp.dot(h, bdad_ref[...], preferred_element_type=jnp.float32)
    as_all = jnp.dot(h, bdas_ref[...], preferred_element_type=jnp.float32)
    as_rows = jnp.transpose(as_all)
    as_maxs = jnp.max(as_rows, axis=1, keepdims=True)

    outs = []
    for i in range(_HEADS):
        g_aug = jnp.concatenate(
            [h[:, i * _HID:(i + 1) * _HID], ones_col], axis=1)
        num, den = _attend(g_aug, ad_all[:, i:i + 1],
                           as_rows[i:i + 1, :], as_maxs[i:i + 1, :])
        outs.append(num / (den + 1e-16))
    h1 = jnp.concatenate(outs, axis=1) + b1_ref[...]
    h1 = jnp.maximum(h1, 0.0)

    g = jnp.dot(h1, w2_ref[...], preferred_element_type=jnp.float32)
    ad2 = jnp.dot(g, ad2_ref[...], preferred_element_type=jnp.float32)
    as2 = jnp.dot(g, as2_ref[...], preferred_element_type=jnp.float32)
    as2_row = jnp.transpose(as2)
    as2_max = jnp.max(as2_row, axis=1, keepdims=True)
    g_aug = jnp.concatenate([g, ones_col], axis=1)
    num, den = _attend(g_aug, ad2, as2_row, as2_max)
    o_ref[...] = num / (den + 1e-16) + b2_ref[...]


def kernel(x, W1, a_src1, a_dst1, b1, W2, a_src2, a_dst2, b2):
    nh = _HEADS * _HID
    blk = (jnp.arange(nh)[:, None] // _HID ==
           jnp.arange(_HEADS)[None, :]).astype(jnp.float32)
    bd_as = blk * (_LOG2E * a_src1).reshape(nh, 1)
    bd_ad = blk * (_LOG2E * a_dst1).reshape(nh, 1)
    return pl.pallas_call(
        _gat_body,
        out_shape=jax.ShapeDtypeStruct((_N, _XD), jnp.float32),
    )(x, W1, bd_as, bd_ad, b1.reshape(1, -1),
      W2, (_LOG2E * a_src2).reshape(_XD, 1), (_LOG2E * a_dst2).reshape(_XD, 1),
      b2.reshape(1, -1))
